# baseline (speedup 1.0000x reference)
import functools

import jax
import jax.numpy as jnp
from jax.experimental import pallas as pl
from jax.experimental.pallas import tpu as pltpu

S = 256
D = 768
L = 48
K = 4
NBUF = 5


def _fused_kernel(x_ref, gate_w_ref, w_hbm, out_w_ref, out_b_ref,
                  ln_g_ref, ln_b_ref, o_ref, g_ref, acc_ref, wbuf, sem):
    l = pl.program_id(0)

    def _copy_a(i, slot):
        return pltpu.make_async_copy(
            w_hbm.at[i, 0:D // 2], wbuf.at[slot, 0:D // 2], sem.at[slot, 0])

    def _copy_b(i, slot):
        return pltpu.make_async_copy(
            w_hbm.at[i, D // 2:D], wbuf.at[slot, D // 2:D], sem.at[slot, 1])

    def _start(i, slot):
        _copy_a(i, slot).start()
        _copy_b(i, slot).start()

    def _wait(i, slot):
        _copy_a(i, slot).wait()
        _copy_b(i, slot).wait()

    @pl.when(l == 0)
    def _prologue():
        for i in range(NBUF):
            _start(i, i)

        x = x_ref[...]
        logits = jax.lax.dot_general(
            x, gate_w_ref[...], (((1,), (1,)), ((), ())),
            preferred_element_type=jnp.float32)
        lane = jax.lax.broadcasted_iota(jnp.int32, (S, L), 1)
        work = logits
        neg_inf = jnp.float32(-jnp.inf)
        vals = []
        sels = []
        for _ in range(K):
            m = jnp.max(work, axis=-1, keepdims=True)
            is_m = work >= m
            first = jnp.min(jnp.where(is_m, lane, L), axis=-1,
                            keepdims=True)
            sel = lane == first
            vals.append(m)
            sels.append(sel)
            work = jnp.where(sel, neg_inf, work)
        v = jnp.concatenate(vals, axis=-1)
        mx = jnp.max(v, axis=-1, keepdims=True)
        e = jnp.exp(v - mx)
        p = e / jnp.sum(e, axis=-1, keepdims=True)
        g = jnp.zeros((S, L), jnp.float32)
        for j in range(K):
            g = g + jnp.where(sels[j], p[:, j:j + 1], 0.0)
        g_ref[...] = g
        acc_ref[...] = jnp.zeros((S, D), jnp.float32)

    slot = jax.lax.rem(l, NBUF)
    _wait(l, slot)

    lane = jax.lax.broadcasted_iota(jnp.int32, (S, L), 1)
    g_col = jnp.sum(jnp.where(lane == l, g_ref[...], 0.0), axis=-1,
                    keepdims=True)
    y = jax.lax.dot_general(
        x_ref[...], wbuf[slot], (((1,), (0,)), ((), ())),
        preferred_element_type=jnp.float32,
        precision=jax.lax.Precision.DEFAULT)
    acc_ref[...] += g_col * y

    @pl.when(l + NBUF < L)
    def _refill():
        _start(l + NBUF, slot)

    @pl.when(l == L - 1)
    def _epilogue():
        x = x_ref[...]
        h = x + jax.lax.dot_general(
            acc_ref[...], out_w_ref[...], (((1,), (1,)), ((), ())),
            preferred_element_type=jnp.float32) + out_b_ref[...]
        mean = jnp.mean(h, axis=-1, keepdims=True)
        c = h - mean
        var = jnp.mean(c * c, axis=-1, keepdims=True)
        o_ref[...] = c * jax.lax.rsqrt(var + 1e-5) * ln_g_ref[...] + ln_b_ref[...]


@functools.partial(jax.jit, static_argnames=())
def kernel(x, gate_w, lattice_weights, out_w, out_b, ln_gamma, ln_beta):
    x2 = x.reshape(S, D)
    out = pl.pallas_call(
        _fused_kernel,
        grid=(L,),
        in_specs=[
            pl.BlockSpec((S, D), lambda l: (0, 0)),
            pl.BlockSpec((L, D), lambda l: (0, 0)),
            pl.BlockSpec(memory_space=pl.ANY),
            pl.BlockSpec((D, D), lambda l: (0, 0)),
            pl.BlockSpec((1, D), lambda l: (0, 0)),
            pl.BlockSpec((1, D), lambda l: (0, 0)),
            pl.BlockSpec((1, D), lambda l: (0, 0)),
        ],
        out_specs=pl.BlockSpec((S, D), lambda l: (0, 0)),
        out_shape=jax.ShapeDtypeStruct((S, D), jnp.float32),
        scratch_shapes=[
            pltpu.VMEM((S, L), jnp.float32),
            pltpu.VMEM((S, D), jnp.float32),
            pltpu.VMEM((NBUF, D, D), jnp.float32),
            pltpu.SemaphoreType.DMA((NBUF, 2)),
        ],
        compiler_params=pltpu.CompilerParams(
            dimension_semantics=("arbitrary",),
        ),
    )(x2, gate_w, lattice_weights, out_w, out_b.reshape(1, D),
      ln_gamma.reshape(1, D), ln_beta.reshape(1, D))
    return out.reshape(1, S, D)


# speedup vs baseline: 1.0222x; 1.0222x over previous
import functools

import jax
import jax.numpy as jnp
from jax.experimental import pallas as pl
from jax.experimental.pallas import tpu as pltpu

S = 256
D = 768
L = 48
K = 4
NBUF = 4


def _fused_kernel(x_ref, gate_w_ref, w_hbm, out_w_ref, out_b_ref,
                  ln_g_ref, ln_b_ref, o_ref, g_ref, acc_ref, wbuf, sem):
    l = pl.program_id(0)

    def _copy_a(i, slot):
        return pltpu.make_async_copy(
            w_hbm.at[i, 0:D // 2], wbuf.at[slot, 0:D // 2], sem.at[slot, 0])

    def _copy_b(i, slot):
        return pltpu.make_async_copy(
            w_hbm.at[i, D // 2:D], wbuf.at[slot, D // 2:D], sem.at[slot, 1])

    def _start(i, slot):
        _copy_a(i, slot).start()
        _copy_b(i, slot).start()

    def _wait(i, slot):
        _copy_a(i, slot).wait()
        _copy_b(i, slot).wait()

    @pl.when(l == 0)
    def _prologue():
        for i in range(NBUF):
            _start(i, i)

        x = x_ref[...]
        logits = jax.lax.dot_general(
            x, gate_w_ref[...], (((1,), (1,)), ((), ())),
            preferred_element_type=jnp.float32)
        lane = jax.lax.broadcasted_iota(jnp.int32, (S, L), 1)
        work = logits
        neg_inf = jnp.float32(-jnp.inf)
        vals = []
        sels = []
        for _ in range(K):
            m = jnp.max(work, axis=-1, keepdims=True)
            is_m = work >= m
            first = jnp.min(jnp.where(is_m, lane, L), axis=-1,
                            keepdims=True)
            sel = lane == first
            vals.append(m)
            sels.append(sel)
            work = jnp.where(sel, neg_inf, work)
        v = jnp.concatenate(vals, axis=-1)
        mx = jnp.max(v, axis=-1, keepdims=True)
        e = jnp.exp(v - mx)
        p = e / jnp.sum(e, axis=-1, keepdims=True)
        g = jnp.zeros((S, L), jnp.float32)
        for j in range(K):
            g = g + jnp.where(sels[j], p[:, j:j + 1], 0.0)
        g_ref[...] = g
        acc_ref[...] = jnp.zeros((S, D), jnp.float32)

    slot = jax.lax.rem(l, NBUF)
    _wait(l, slot)

    lane = jax.lax.broadcasted_iota(jnp.int32, (S, L), 1)
    g_col = jnp.sum(jnp.where(lane == l, g_ref[...], 0.0), axis=-1,
                    keepdims=True)
    y = jax.lax.dot_general(
        x_ref[...], wbuf[slot], (((1,), (0,)), ((), ())),
        preferred_element_type=jnp.float32,
        precision=jax.lax.Precision.DEFAULT)
    acc_ref[...] += g_col * y

    @pl.when(l + NBUF < L)
    def _refill():
        _start(l + NBUF, slot)

    @pl.when(l == L - 1)
    def _epilogue():
        x = x_ref[...]
        h = x + jax.lax.dot_general(
            acc_ref[...], out_w_ref[...], (((1,), (1,)), ((), ())),
            preferred_element_type=jnp.float32) + out_b_ref[...]
        mean = jnp.mean(h, axis=-1, keepdims=True)
        c = h - mean
        var = jnp.mean(c * c, axis=-1, keepdims=True)
        o_ref[...] = c * jax.lax.rsqrt(var + 1e-5) * ln_g_ref[...] + ln_b_ref[...]


@functools.partial(jax.jit, static_argnames=())
def kernel(x, gate_w, lattice_weights, out_w, out_b, ln_gamma, ln_beta):
    x2 = x.reshape(S, D)
    out = pl.pallas_call(
        _fused_kernel,
        grid=(L,),
        in_specs=[
            pl.BlockSpec((S, D), lambda l: (0, 0)),
            pl.BlockSpec((L, D), lambda l: (0, 0)),
            pl.BlockSpec(memory_space=pl.ANY),
            pl.BlockSpec((D, D), lambda l: (0, 0)),
            pl.BlockSpec((1, D), lambda l: (0, 0)),
            pl.BlockSpec((1, D), lambda l: (0, 0)),
            pl.BlockSpec((1, D), lambda l: (0, 0)),
        ],
        out_specs=pl.BlockSpec((S, D), lambda l: (0, 0)),
        out_shape=jax.ShapeDtypeStruct((S, D), jnp.float32),
        scratch_shapes=[
            pltpu.VMEM((S, L), jnp.float32),
            pltpu.VMEM((S, D), jnp.float32),
            pltpu.VMEM((NBUF, D, D), jnp.float32),
            pltpu.SemaphoreType.DMA((NBUF, 2)),
        ],
        compiler_params=pltpu.CompilerParams(
            dimension_semantics=("arbitrary",),
        ),
    )(x2, gate_w, lattice_weights, out_w, out_b.reshape(1, D),
      ln_gamma.reshape(1, D), ln_beta.reshape(1, D))
    return out.reshape(1, S, D)
